# baseline (speedup 1.0000x reference)
import functools

import jax
import jax.numpy as jnp
from jax import lax
from jax.experimental import pallas as pl
from jax.experimental.pallas import tpu as pltpu
from jax.experimental.pallas import tpu_sc as plsc

B, F, D = 4096, 100, 128
NC, NS = 2, 16
NW = NC * NS
NRG = 16
RPW = B // NRG
FPW = F // 2
NBUF = 2
G = FPW // NBUF

_mesh = plsc.VectorSubcoreMesh(core_axis_name="c", subcore_axis_name="s")


@functools.partial(
    pl.kernel,
    mesh=_mesh,
    out_type=jax.ShapeDtypeStruct((F, B, D), jnp.float32),
    scratch_types=(
        [pltpu.VMEM((RPW, D), jnp.float32) for _ in range(NBUF)]
        + [pltpu.SemaphoreType.DMA for _ in range(2 * NBUF)]
    ),
)
def _rev_copy(x_hbm, out_hbm, *refs):
    buf = refs[0:NBUF]
    rsem = refs[NBUF:2 * NBUF]
    wsem = refs[2 * NBUF:3 * NBUF]
    wid = lax.axis_index("s") * NC + lax.axis_index("c")
    r0 = (wid % NRG) * RPW
    fpar = wid // NRG

    def start_read(k, j):
        f = fpar + 2 * j
        pltpu.async_copy(x_hbm.at[F - 1 - f].at[pl.ds(r0, RPW)],
                         buf[k], rsem[k])

    def wait_read(k):
        pltpu.make_async_copy(x_hbm.at[0].at[pl.ds(r0, RPW)],
                              buf[k], rsem[k]).wait()

    def start_write(k, j):
        f = fpar + 2 * j
        pltpu.async_copy(buf[k], out_hbm.at[f].at[pl.ds(r0, RPW)], wsem[k])

    def wait_write(k):
        pltpu.make_async_copy(buf[k], out_hbm.at[0].at[pl.ds(r0, RPW)],
                              wsem[k]).wait()

    for k in range(NBUF):
        start_read(k, k)

    def body(g, carry):
        j0 = g * NBUF
        for k in range(NBUF):
            wait_read(k)
            start_write(k, j0 + k)

        @pl.when(g < G - 1)
        def _next():
            for k in range(NBUF):
                wait_write(k)
                start_read(k, j0 + NBUF + k)

        return carry

    lax.fori_loop(0, G, body, 0)
    for k in range(NBUF):
        wait_write(k)


def kernel(x):
    out_t = _rev_copy(x.transpose(1, 0, 2))
    return out_t.transpose(1, 0, 2)


# speedup vs baseline: 1.0270x; 1.0270x over previous
import functools

import jax
import jax.numpy as jnp
from jax import lax
from jax.experimental import pallas as pl
from jax.experimental.pallas import tpu as pltpu
from jax.experimental.pallas import tpu_sc as plsc

B, F, D = 4096, 100, 128
NC, NS = 2, 16
NW = NC * NS
RPW = B // NW
NSLOT = 2
G = F // (2 * NSLOT)

_mesh = plsc.VectorSubcoreMesh(core_axis_name="c", subcore_axis_name="s")


@functools.partial(
    pl.kernel,
    mesh=_mesh,
    out_type=jax.ShapeDtypeStruct((F, B, D), jnp.float32),
    scratch_types=(
        [pltpu.VMEM((RPW, D), jnp.float32) for _ in range(2 * NSLOT)]
        + [pltpu.SemaphoreType.DMA for _ in range(4 * NSLOT)]
    ),
)
def _rev_copy(x_hbm, out_hbm, *refs):
    buf = refs[0:2 * NSLOT]
    rsem = refs[2 * NSLOT:4 * NSLOT]
    wsem = refs[4 * NSLOT:6 * NSLOT]
    wid = lax.axis_index("s") * NC + lax.axis_index("c")
    r0 = wid * RPW

    def start_read(k, f):
        pltpu.async_copy(x_hbm.at[F - 1 - f].at[pl.ds(r0, RPW)],
                         buf[k], rsem[k])

    def wait_read(k):
        pltpu.make_async_copy(x_hbm.at[0].at[pl.ds(r0, RPW)],
                              buf[k], rsem[k]).wait()

    def start_write(k, f):
        pltpu.async_copy(buf[k], out_hbm.at[f].at[pl.ds(r0, RPW)], wsem[k])

    def wait_write(k):
        pltpu.make_async_copy(buf[k], out_hbm.at[0].at[pl.ds(r0, RPW)],
                              wsem[k]).wait()

    A = list(range(NSLOT))
    Bset = list(range(NSLOT, 2 * NSLOT))

    for i, k in enumerate(A):
        start_read(k, i)

    def body(h, carry):
        f0 = h * 2 * NSLOT
        for k in A:
            wait_read(k)

        @pl.when(h > 0)
        def _wb():
            for k in Bset:
                wait_write(k)

        for i, k in enumerate(A):
            start_write(k, f0 + i)
        for i, k in enumerate(Bset):
            start_read(k, f0 + NSLOT + i)
        for k in Bset:
            wait_read(k)
        for k in A:
            wait_write(k)
        for i, k in enumerate(Bset):
            start_write(k, f0 + NSLOT + i)

        @pl.when(h < G - 1)
        def _ra():
            for i, k in enumerate(A):
                start_read(k, f0 + 2 * NSLOT + i)

        return carry

    lax.fori_loop(0, G, body, 0)
    for k in Bset:
        wait_write(k)


def kernel(x):
    out_t = _rev_copy(x.transpose(1, 0, 2))
    return out_t.transpose(1, 0, 2)


# speedup vs baseline: 1.1076x; 1.0785x over previous
import functools

import jax
import jax.numpy as jnp
from jax import lax
from jax.experimental import pallas as pl
from jax.experimental.pallas import tpu as pltpu
from jax.experimental.pallas import tpu_sc as plsc

B, F, D = 4096, 100, 128
NC, NS = 2, 16
NW = NC * NS
RPW = B // NW
NSLOT = 2
G = F // (2 * NSLOT)

_mesh = plsc.VectorSubcoreMesh(core_axis_name="c", subcore_axis_name="s")


@functools.partial(
    pl.kernel,
    mesh=_mesh,
    out_type=jax.ShapeDtypeStruct((F, B, D), jnp.float32),
    scratch_types=(
        [pltpu.VMEM_SHARED((NS, 2 * NSLOT, RPW, D), jnp.float32)]
        + [pltpu.SemaphoreType.DMA for _ in range(4 * NSLOT)]
    ),
)
def _rev_copy(x_hbm, out_hbm, shared, *refs):
    rsem = refs[0:2 * NSLOT]
    wsem = refs[2 * NSLOT:4 * NSLOT]
    sid = lax.axis_index("s")
    wid = sid * NC + lax.axis_index("c")
    r0 = wid * RPW

    def start_read(k, f):
        pltpu.async_copy(x_hbm.at[F - 1 - f].at[pl.ds(r0, RPW)],
                         shared.at[sid, k], rsem[k])

    def wait_read(k):
        pltpu.make_async_copy(x_hbm.at[0].at[pl.ds(r0, RPW)],
                              shared.at[sid, k], rsem[k]).wait()

    def start_write(k, f):
        pltpu.async_copy(shared.at[sid, k], out_hbm.at[f].at[pl.ds(r0, RPW)],
                         wsem[k])

    def wait_write(k):
        pltpu.make_async_copy(shared.at[sid, k], out_hbm.at[0].at[pl.ds(r0, RPW)],
                              wsem[k]).wait()

    A = list(range(NSLOT))
    Bset = list(range(NSLOT, 2 * NSLOT))

    for i, k in enumerate(A):
        start_read(k, i)

    def body(h, carry):
        f0 = h * 2 * NSLOT
        for k in A:
            wait_read(k)

        @pl.when(h > 0)
        def _wb():
            for k in Bset:
                wait_write(k)

        for i, k in enumerate(A):
            start_write(k, f0 + i)
        for i, k in enumerate(Bset):
            start_read(k, f0 + NSLOT + i)
        for k in Bset:
            wait_read(k)
        for k in A:
            wait_write(k)
        for i, k in enumerate(Bset):
            start_write(k, f0 + NSLOT + i)

        @pl.when(h < G - 1)
        def _ra():
            for i, k in enumerate(A):
                start_read(k, f0 + 2 * NSLOT + i)

        return carry

    lax.fori_loop(0, G, body, 0)
    for k in Bset:
        wait_write(k)


def kernel(x):
    out_t = _rev_copy(x.transpose(1, 0, 2))
    return out_t.transpose(1, 0, 2)
